# Initial kernel scaffold; baseline (speedup 1.0000x reference)
import jax
import jax.numpy as jnp
from jax.experimental import pallas as pl


def kernel(rpn_box, rpn_prob, anchors):
    raise NotImplementedError("write your pallas kernel here")


# speedup vs baseline: 13.1221x; 13.1221x over previous
import jax
import jax.numpy as jnp
from jax.experimental import pallas as pl
from jax.experimental.pallas import tpu as pltpu

_TOP_N = 2000
_TOP_N_POST = 300
_NMS_THRESH = 0.7
_ROWS = 16
_COLS = 128
_PAD = _ROWS * _COLS


def _decode_nms_body(dx_ref, dy_ref, dw_ref, dh_ref,
                     ax1_ref, ay1_ref, ax2_ref, ay2_ref,
                     x1_o, y1_o, x2_o, y2_o, keep_o,
                     area_s):
    ax1 = ax1_ref[...]
    ay1 = ay1_ref[...]
    ax2 = ax2_ref[...]
    ay2 = ay2_ref[...]
    aw = ax2 - ax1 + 1.0
    ah = ay2 - ay1 + 1.0
    acx = ax1 + 0.5 * aw
    acy = ay1 + 0.5 * ah
    cx = dx_ref[...] * aw + acx
    cy = dy_ref[...] * ah + acy
    w = jnp.exp(jnp.clip(dw_ref[...], -10.0, 10.0)) * aw
    h = jnp.exp(jnp.clip(dh_ref[...], -10.0, 10.0)) * ah
    x1 = cx - 0.5 * w
    y1 = cy - 0.5 * h
    x2 = cx + 0.5 * w
    y2 = cy + 0.5 * h
    x1_o[...] = x1
    y1_o[...] = y1
    x2_o[...] = x2
    y2_o[...] = y2
    area = jnp.maximum(x2 - x1, 0.0) * jnp.maximum(y2 - y1, 0.0)
    area_s[...] = area
    keep_o[...] = jnp.ones((_ROWS, _COLS), jnp.float32)

    lane = jax.lax.broadcasted_iota(jnp.int32, (1, _COLS), 1)
    flat_idx = (jax.lax.broadcasted_iota(jnp.int32, (_ROWS, _COLS), 0) * _COLS
                + jax.lax.broadcasted_iota(jnp.int32, (_ROWS, _COLS), 1))

    def body(i, _):
        r = i // _COLS
        c = i - r * _COLS
        sel = lane == c

        def pick(ref):
            row = ref[pl.ds(r, 1), :]
            return jnp.sum(jnp.where(sel, row, 0.0))

        bx1 = pick(x1_o)
        by1 = pick(y1_o)
        bx2 = pick(x2_o)
        by2 = pick(y2_o)
        barea = pick(area_s)
        bkeep = pick(keep_o)

        ix1 = jnp.maximum(bx1, x1)
        iy1 = jnp.maximum(by1, y1)
        ix2 = jnp.minimum(bx2, x2)
        iy2 = jnp.minimum(by2, y2)
        iw = jnp.maximum(ix2 - ix1, 0.0)
        ih = jnp.maximum(iy2 - iy1, 0.0)
        inter = iw * ih
        union = barea + area - inter
        iou = inter / (union + 1e-8)
        sup = (iou > _NMS_THRESH) & (flat_idx > i) & (bkeep > 0.5)
        keep_o[...] = jnp.where(sup, 0.0, keep_o[...])
        return 0

    jax.lax.fori_loop(0, _TOP_N, body, 0, unroll=False)


def _pad_col(v):
    return jnp.pad(v, (0, _PAD - _TOP_N)).reshape(_ROWS, _COLS)


@jax.jit
def kernel(rpn_box, rpn_prob, anchors):
    objness = 1.0 - rpn_prob[:, 0]
    scores, inds = jax.lax.top_k(objness, _TOP_N)
    sel_box = jnp.take(rpn_box, inds, axis=0)
    sel_anc = jnp.take(anchors, inds, axis=0)

    ins = [_pad_col(sel_box[:, j]) for j in range(4)]
    ins += [_pad_col(sel_anc[:, j]) for j in range(4)]

    shp = jax.ShapeDtypeStruct((_ROWS, _COLS), jnp.float32)
    x1, y1, x2, y2, keepf = pl.pallas_call(
        _decode_nms_body,
        out_shape=[shp] * 5,
        scratch_shapes=[pltpu.VMEM((_ROWS, _COLS), jnp.float32)],
    )(*ins)

    decoded = jnp.stack(
        [x1.reshape(-1), y1.reshape(-1), x2.reshape(-1), y2.reshape(-1)],
        axis=1)[:_TOP_N]
    keep = keepf.reshape(-1)[:_TOP_N] > 0.5
    masked = jnp.where(keep, scores, -1e9)
    final_scores, fi = jax.lax.top_k(masked, _TOP_N_POST)
    final_boxes = jnp.take(decoded, fi, axis=0)
    return jnp.concatenate([final_boxes, final_scores[:, None]], axis=1)
